# initial kernel scaffold (unmeasured)
import jax
import jax.numpy as jnp
from jax import lax
from jax.experimental import pallas as pl
from jax.experimental.pallas import tpu as pltpu

N_DEV = 16


def kernel(x, w_mat):
    m, k = x.shape
    k2, n = w_mat.shape
    assert k == k2
    ch = m // N_DEV

    def body(x_ref, w_ref, out_ref, comm_ref, ss_rs, rs_rs, ss_ag, rs_ag):
        my = lax.axis_index("i")
        left = jnp.mod(my - 1, N_DEV)
        right = jnp.mod(my + 1, N_DEV)

        barrier_sem = pltpu.get_barrier_semaphore()
        for nbr in (left, right):
            pl.semaphore_signal(
                barrier_sem, inc=1,
                device_id=(nbr,), device_id_type=pl.DeviceIdType.MESH,
            )
        pl.semaphore_wait(barrier_sem, 2)

        for c in range(N_DEV):
            out_ref[pl.ds(c * ch, ch), :] = jnp.dot(
                x_ref[pl.ds(c * ch, ch), :], w_ref[:, :],
                preferred_element_type=jnp.float32,
            )

        for h in range(N_DEV - 1):
            slot = h % 2
            s_idx = jnp.mod(my - h, N_DEV)
            r_idx = jnp.mod(my - h - 1, N_DEV)
            rdma = pltpu.make_async_remote_copy(
                src_ref=out_ref.at[pl.ds(s_idx * ch, ch), :],
                dst_ref=comm_ref.at[slot],
                send_sem=ss_rs.at[slot],
                recv_sem=rs_rs.at[slot],
                device_id=(right,),
                device_id_type=pl.DeviceIdType.MESH,
            )
            rdma.start()
            rdma.wait()
            out_ref[pl.ds(r_idx * ch, ch), :] = (
                out_ref[pl.ds(r_idx * ch, ch), :] + comm_ref[slot]
            )

        for h in range(N_DEV - 1):
            slot = h % 2
            g_idx = jnp.mod(my + 1 - h, N_DEV)
            rdma = pltpu.make_async_remote_copy(
                src_ref=out_ref.at[pl.ds(g_idx * ch, ch), :],
                dst_ref=out_ref.at[pl.ds(g_idx * ch, ch), :],
                send_sem=ss_ag.at[slot],
                recv_sem=rs_ag.at[slot],
                device_id=(right,),
                device_id_type=pl.DeviceIdType.MESH,
            )
            rdma.start()
            rdma.wait()

        amax = jnp.float32(0.0)
        for c in range(N_DEV):
            blk = jnp.maximum(out_ref[pl.ds(c * ch, ch), :], 0.0)
            out_ref[pl.ds(c * ch, ch), :] = blk
            amax = jnp.maximum(amax, jnp.max(blk))
        inv = 127.0 / amax
        scale = amax / 127.0
        for c in range(N_DEV):
            blk = out_ref[pl.ds(c * ch, ch), :]
            q = jnp.clip(jnp.round(blk * inv), -127.0, 127.0)
            out_ref[pl.ds(c * ch, ch), :] = q * scale

    return pl.pallas_call(
        body,
        out_shape=jax.ShapeDtypeStruct((m, n), jnp.float32),
        in_specs=[
            pl.BlockSpec(memory_space=pltpu.VMEM),
            pl.BlockSpec(memory_space=pltpu.VMEM),
        ],
        out_specs=pl.BlockSpec(memory_space=pltpu.VMEM),
        scratch_shapes=[
            pltpu.VMEM((2, ch, n), jnp.float32),
            pltpu.SemaphoreType.DMA((2,)),
            pltpu.SemaphoreType.DMA((2,)),
            pltpu.SemaphoreType.DMA((2,)),
            pltpu.SemaphoreType.DMA((2,)),
        ],
        compiler_params=pltpu.CompilerParams(collective_id=0),
    )(x, w_mat)


# baseline (device time: 788979 ns/iter reference)
import jax
import jax.numpy as jnp
from jax import lax
from jax.experimental import pallas as pl
from jax.experimental.pallas import tpu as pltpu

N_DEV = 16


def kernel(x, w_mat):
    m, k = x.shape
    k2, n = w_mat.shape
    assert k == k2
    ch = m // N_DEV

    def body(x_ref, w_ref, out_ref, comm_ref, ss_rs, rs_rs, ss_ag, rs_ag):
        my = lax.axis_index("i")
        left = jnp.mod(my - 1, N_DEV)
        right = jnp.mod(my + 1, N_DEV)

        barrier_sem = pltpu.get_barrier_semaphore()
        for nbr in (left, right):
            pl.semaphore_signal(
                barrier_sem, inc=1,
                device_id=(nbr,), device_id_type=pl.DeviceIdType.MESH,
            )
        pl.semaphore_wait(barrier_sem, 2)

        for c in range(N_DEV):
            out_ref[pl.ds(c * ch, ch), :] = jnp.dot(
                x_ref[pl.ds(c * ch, ch), :], w_ref[:, :],
                preferred_element_type=jnp.float32,
            )

        for h in range(N_DEV - 1):
            slot = h % 2
            s_idx = jnp.mod(my - h, N_DEV)
            r_idx = jnp.mod(my - h - 1, N_DEV)
            rdma = pltpu.make_async_remote_copy(
                src_ref=out_ref.at[pl.ds(s_idx * ch, ch), :],
                dst_ref=comm_ref.at[slot],
                send_sem=ss_rs.at[slot],
                recv_sem=rs_rs.at[slot],
                device_id=(right,),
                device_id_type=pl.DeviceIdType.MESH,
            )
            rdma.start()
            rdma.wait()
            out_ref[pl.ds(r_idx * ch, ch), :] = (
                out_ref[pl.ds(r_idx * ch, ch), :] + comm_ref[slot]
            )

        for h in range(N_DEV - 1):
            slot = h % 2
            g_idx = jnp.mod(my + 1 - h, N_DEV)
            rdma = pltpu.make_async_remote_copy(
                src_ref=out_ref.at[pl.ds(g_idx * ch, ch), :],
                dst_ref=out_ref.at[pl.ds(g_idx * ch, ch), :],
                send_sem=ss_ag.at[slot],
                recv_sem=rs_ag.at[slot],
                device_id=(right,),
                device_id_type=pl.DeviceIdType.MESH,
            )
            rdma.start()
            rdma.wait()

        amax = jnp.float32(0.0)
        for c in range(N_DEV):
            blk = jnp.maximum(out_ref[pl.ds(c * ch, ch), :], 0.0)
            out_ref[pl.ds(c * ch, ch), :] = blk
            amax = jnp.maximum(amax, jnp.max(blk))
        inv = 127.0 / amax
        scale = amax / 127.0
        for c in range(N_DEV):
            blk = out_ref[pl.ds(c * ch, ch), :]
            q = jnp.clip(jnp.round(blk * inv), -127.0, 127.0)
            out_ref[pl.ds(c * ch, ch), :] = q * scale

    return pl.pallas_call(
        body,
        out_shape=jax.ShapeDtypeStruct((m, n), jnp.float32),
        in_specs=[
            pl.BlockSpec(memory_space=pltpu.VMEM),
            pl.BlockSpec(memory_space=pltpu.VMEM),
        ],
        out_specs=pl.BlockSpec(memory_space=pltpu.VMEM),
        scratch_shapes=[
            pltpu.VMEM((2, ch, n), jnp.float32),
            pltpu.SemaphoreType.DMA((2,)),
            pltpu.SemaphoreType.DMA((2,)),
            pltpu.SemaphoreType.DMA((2,)),
            pltpu.SemaphoreType.DMA((2,)),
        ],
        compiler_params=pltpu.CompilerParams(
            collective_id=0,
            vmem_limit_bytes=64 * 1024 * 1024,
        ),
    )(x, w_mat)


# device time: 326963 ns/iter; 2.4131x vs baseline; 2.4131x over previous
import jax
import jax.numpy as jnp
from jax import lax
from jax.experimental import pallas as pl
from jax.experimental.pallas import tpu as pltpu

N_DEV = 16

RING = [0, 1, 5, 9, 13, 14, 10, 6, 2, 3, 7, 11, 15, 12, 8, 4]
POS = [0] * N_DEV
for _p, _id in enumerate(RING):
    POS[_id] = _p


def kernel(x, w_mat):
    m, k = x.shape
    k2, n = w_mat.shape
    assert k == k2
    ch = m // N_DEV
    n2 = n // 2

    my = lax.axis_index("i")
    ring = jnp.asarray(RING, dtype=jnp.int32)
    rp = jnp.asarray(POS, dtype=jnp.int32)[my]
    right = ring[jnp.mod(rp + 1, N_DEV)]
    left = ring[jnp.mod(rp - 1, N_DEV)]
    params = jnp.stack([rp, left, right]).astype(jnp.int32)

    def body(params_ref, x_ref, w_ref, out_ref,
             comm_cw, comm_cc, q_cw, q_cc, ag_cw, ag_cc, amax_ref,
             rs_s_cw, rs_r_cw, rs_s_cc, rs_r_cc,
             ag_s_cw, ag_r_cw, ag_s_cc, ag_r_cc,
             ax_s, ax_r):
        rp = params_ref[0]
        left = params_ref[1]
        right = params_ref[2]

        def rows(idx):
            return pl.ds(idx * ch, ch)

        L = pl.ds(0, n2)
        R = pl.ds(n2, n2)

        barrier_sem = pltpu.get_barrier_semaphore()
        for nbr in (left, right):
            pl.semaphore_signal(
                barrier_sem, inc=1,
                device_id=(nbr,), device_id_type=pl.DeviceIdType.MESH,
            )
        pl.semaphore_wait(barrier_sem, 2)

        for c in range(N_DEV):
            out_ref[rows(c), :] = jnp.dot(
                x_ref[pl.ds(c * ch, ch), :], w_ref[:, :],
                preferred_element_type=jnp.float32,
            )

        for h in range(N_DEV - 1):
            slot = h % 2
            s_cw = jnp.mod(rp - h, N_DEV)
            r_cw = jnp.mod(rp - h - 1, N_DEV)
            s_cc = jnp.mod(rp + h, N_DEV)
            r_cc = jnp.mod(rp + h + 1, N_DEV)
            rd_cw = pltpu.make_async_remote_copy(
                src_ref=out_ref.at[rows(s_cw), L],
                dst_ref=comm_cw.at[slot],
                send_sem=rs_s_cw.at[slot], recv_sem=rs_r_cw.at[slot],
                device_id=(right,), device_id_type=pl.DeviceIdType.MESH,
            )
            rd_cc = pltpu.make_async_remote_copy(
                src_ref=out_ref.at[rows(s_cc), R],
                dst_ref=comm_cc.at[slot],
                send_sem=rs_s_cc.at[slot], recv_sem=rs_r_cc.at[slot],
                device_id=(left,), device_id_type=pl.DeviceIdType.MESH,
            )
            rd_cw.start()
            rd_cc.start()
            rd_cw.wait()
            rd_cc.wait()
            out_ref[rows(r_cw), L] = out_ref[rows(r_cw), L] + comm_cw[slot]
            out_ref[rows(r_cc), R] = out_ref[rows(r_cc), R] + comm_cc[slot]

        own_cw = jnp.mod(rp + 1, N_DEV)
        own_cc = jnp.mod(rp - 1, N_DEV)

        blk_cw = jnp.maximum(out_ref[rows(own_cw), L], 0.0)
        out_ref[rows(own_cw), L] = blk_cw
        blk_cc = jnp.maximum(out_ref[rows(own_cc), R], 0.0)
        out_ref[rows(own_cc), R] = blk_cc
        local_amax = jnp.maximum(jnp.max(blk_cw), jnp.max(blk_cc))

        amax_ref[rp] = jnp.full((8, 128), local_amax, dtype=jnp.float32)
        for p in range(N_DEV):
            @pl.when(p != rp)
            def _():
                rd = pltpu.make_async_remote_copy(
                    src_ref=amax_ref.at[rp],
                    dst_ref=amax_ref.at[rp],
                    send_sem=ax_s.at[p], recv_sem=ax_r.at[rp],
                    device_id=(RING[p],),
                    device_id_type=pl.DeviceIdType.MESH,
                )
                rd.start()
        for p in range(N_DEV):
            @pl.when(p != rp)
            def _():
                dummy = pltpu.make_async_remote_copy(
                    src_ref=amax_ref.at[p], dst_ref=amax_ref.at[p],
                    send_sem=ax_s.at[p], recv_sem=ax_r.at[p],
                    device_id=(0,), device_id_type=pl.DeviceIdType.MESH,
                )
                dummy.wait_send()
                dummy.wait_recv()
        g_amax = jnp.max(amax_ref[:, :, :])
        inv = 127.0 / g_amax
        scale = g_amax / 127.0

        q_cw[:, :] = jnp.clip(
            jnp.round(blk_cw * inv), -127.0, 127.0).astype(jnp.int8)
        out_ref[rows(own_cw), L] = q_cw[:, :].astype(jnp.float32) * scale
        q_cc[:, :] = jnp.clip(
            jnp.round(blk_cc * inv), -127.0, 127.0).astype(jnp.int8)
        out_ref[rows(own_cc), R] = q_cc[:, :].astype(jnp.float32) * scale

        for h in range(N_DEV - 1):
            slot = h % 2
            rd_cw = pltpu.make_async_remote_copy(
                src_ref=(q_cw if h == 0 else ag_cw.at[(h - 1) % 2]),
                dst_ref=ag_cw.at[slot],
                send_sem=ag_s_cw.at[slot], recv_sem=ag_r_cw.at[slot],
                device_id=(right,), device_id_type=pl.DeviceIdType.MESH,
            )
            rd_cc = pltpu.make_async_remote_copy(
                src_ref=(q_cc if h == 0 else ag_cc.at[(h - 1) % 2]),
                dst_ref=ag_cc.at[slot],
                send_sem=ag_s_cc.at[slot], recv_sem=ag_r_cc.at[slot],
                device_id=(left,), device_id_type=pl.DeviceIdType.MESH,
            )
            rd_cw.start()
            rd_cc.start()
            rd_cw.wait()
            rd_cc.wait()
            c_cw = jnp.mod(rp - h, N_DEV)
            c_cc = jnp.mod(rp + h, N_DEV)
            out_ref[rows(c_cw), L] = ag_cw[slot].astype(jnp.float32) * scale
            out_ref[rows(c_cc), R] = ag_cc[slot].astype(jnp.float32) * scale

    return pl.pallas_call(
        body,
        out_shape=jax.ShapeDtypeStruct((m, n), jnp.float32),
        in_specs=[
            pl.BlockSpec(memory_space=pltpu.SMEM),
            pl.BlockSpec(memory_space=pltpu.VMEM),
            pl.BlockSpec(memory_space=pltpu.VMEM),
        ],
        out_specs=pl.BlockSpec(memory_space=pltpu.VMEM),
        scratch_shapes=[
            pltpu.VMEM((2, ch, n2), jnp.float32),
            pltpu.VMEM((2, ch, n2), jnp.float32),
            pltpu.VMEM((ch, n2), jnp.int8),
            pltpu.VMEM((ch, n2), jnp.int8),
            pltpu.VMEM((2, ch, n2), jnp.int8),
            pltpu.VMEM((2, ch, n2), jnp.int8),
            pltpu.VMEM((N_DEV, 8, 128), jnp.float32),
            pltpu.SemaphoreType.DMA((2,)),
            pltpu.SemaphoreType.DMA((2,)),
            pltpu.SemaphoreType.DMA((2,)),
            pltpu.SemaphoreType.DMA((2,)),
            pltpu.SemaphoreType.DMA((2,)),
            pltpu.SemaphoreType.DMA((2,)),
            pltpu.SemaphoreType.DMA((2,)),
            pltpu.SemaphoreType.DMA((2,)),
            pltpu.SemaphoreType.DMA((N_DEV,)),
            pltpu.SemaphoreType.DMA((N_DEV,)),
        ],
        compiler_params=pltpu.CompilerParams(
            collective_id=0,
            vmem_limit_bytes=64 * 1024 * 1024,
        ),
    )(params, x, w_mat)


# device time: 273674 ns/iter; 2.8829x vs baseline; 1.1947x over previous
import jax
import jax.numpy as jnp
from jax import lax
from jax.experimental import pallas as pl
from jax.experimental.pallas import tpu as pltpu

N_DEV = 16
K_SUB = 2

RING = [0, 1, 5, 9, 13, 14, 10, 6, 2, 3, 7, 11, 15, 12, 8, 4]
POS = [0] * N_DEV
for _p, _id in enumerate(RING):
    POS[_id] = _p


def kernel(x, w_mat):
    m, k = x.shape
    k2, n = w_mat.shape
    assert k == k2
    ch = m // N_DEV
    sub = ch // K_SUB
    n2 = n // 2

    my = lax.axis_index("i")
    ring = jnp.asarray(RING, dtype=jnp.int32)
    rp = jnp.asarray(POS, dtype=jnp.int32)[my]
    right = ring[jnp.mod(rp + 1, N_DEV)]
    left = ring[jnp.mod(rp - 1, N_DEV)]
    params = jnp.stack([rp, left, right]).astype(jnp.int32)

    def body(params_ref, x_ref, w_ref, out_ref,
             comm_cw, comm_cc, q_cw, q_cc, ag_cw, ag_cc, amax_ref,
             rs_s_cw, rs_r_cw, rs_s_cc, rs_r_cc,
             ag_s_cw, ag_r_cw, ag_s_cc, ag_r_cc,
             ax_s, ax_r):
        rp = params_ref[0]
        left = params_ref[1]
        right = params_ref[2]

        def rows(idx):
            return pl.ds(idx * ch, ch)

        def srows(idx, j):
            return pl.ds(idx * ch + j * sub, sub)

        L = pl.ds(0, n2)
        R = pl.ds(n2, n2)

        def gemm_half(c, half):
            out_ref[rows(c), half] = jnp.dot(
                x_ref[rows(c), :], w_ref[:, half],
                preferred_element_type=jnp.float32,
            )

        def rs_rdma(h, j):
            slot = h % 2
            s_cw = jnp.mod(rp - h, N_DEV)
            s_cc = jnp.mod(rp + h, N_DEV)
            rd_cw = pltpu.make_async_remote_copy(
                src_ref=out_ref.at[srows(s_cw, j), L],
                dst_ref=comm_cw.at[slot, pl.ds(j * sub, sub)],
                send_sem=rs_s_cw.at[slot, j], recv_sem=rs_r_cw.at[slot, j],
                device_id=(right,), device_id_type=pl.DeviceIdType.MESH,
            )
            rd_cc = pltpu.make_async_remote_copy(
                src_ref=out_ref.at[srows(s_cc, j), R],
                dst_ref=comm_cc.at[slot, pl.ds(j * sub, sub)],
                send_sem=rs_s_cc.at[slot, j], recv_sem=rs_r_cc.at[slot, j],
                device_id=(left,), device_id_type=pl.DeviceIdType.MESH,
            )
            return rd_cw, rd_cc

        barrier_sem = pltpu.get_barrier_semaphore()
        for nbr in (left, right):
            pl.semaphore_signal(
                barrier_sem, inc=1,
                device_id=(nbr,), device_id_type=pl.DeviceIdType.MESH,
            )
        pl.semaphore_wait(barrier_sem, 2)

        gemm_half(rp, L)
        gemm_half(rp, R)
        for j in range(K_SUB):
            rd_cw, rd_cc = rs_rdma(0, j)
            rd_cw.start()
            rd_cc.start()
        gemm_half(jnp.mod(rp - 1, N_DEV), L)
        gemm_half(jnp.mod(rp + 1, N_DEV), R)

        for h in range(N_DEV - 1):
            r_cw = jnp.mod(rp - h - 1, N_DEV)
            r_cc = jnp.mod(rp + h + 1, N_DEV)
            slot = h % 2
            for j in range(K_SUB):
                rd_cw, rd_cc = rs_rdma(h, j)
                rd_cw.wait_recv()
                rd_cc.wait_recv()
                cslice = pl.ds(j * sub, sub)
                out_ref[srows(r_cw, j), L] = (
                    out_ref[srows(r_cw, j), L] + comm_cw[slot, cslice]
                )
                out_ref[srows(r_cc, j), R] = (
                    out_ref[srows(r_cc, j), R] + comm_cc[slot, cslice]
                )
                if h < N_DEV - 2:
                    if h >= 1:
                        pv_cw, pv_cc = rs_rdma(h - 1, j)
                        pv_cw.wait_send()
                        pv_cc.wait_send()
                    nx_cw, nx_cc = rs_rdma(h + 1, j)
                    nx_cw.start()
                    nx_cc.start()
            if h < N_DEV - 2:
                gemm_half(jnp.mod(rp - h - 2, N_DEV), L)
                gemm_half(jnp.mod(rp + h + 2, N_DEV), R)
        for h in (N_DEV - 3, N_DEV - 2):
            for j in range(K_SUB):
                rd_cw, rd_cc = rs_rdma(h, j)
                rd_cw.wait_send()
                rd_cc.wait_send()

        own_cw = jnp.mod(rp + 1, N_DEV)
        own_cc = jnp.mod(rp - 1, N_DEV)

        blk_cw = jnp.maximum(out_ref[rows(own_cw), L], 0.0)
        out_ref[rows(own_cw), L] = blk_cw
        blk_cc = jnp.maximum(out_ref[rows(own_cc), R], 0.0)
        out_ref[rows(own_cc), R] = blk_cc
        local_amax = jnp.maximum(jnp.max(blk_cw), jnp.max(blk_cc))

        amax_ref[rp] = jnp.full((8, 128), local_amax, dtype=jnp.float32)
        for p in range(N_DEV):
            @pl.when(p != rp)
            def _():
                rd = pltpu.make_async_remote_copy(
                    src_ref=amax_ref.at[rp],
                    dst_ref=amax_ref.at[rp],
                    send_sem=ax_s.at[p], recv_sem=ax_r.at[rp],
                    device_id=(RING[p],),
                    device_id_type=pl.DeviceIdType.MESH,
                )
                rd.start()
        for p in range(N_DEV):
            @pl.when(p != rp)
            def _():
                dummy = pltpu.make_async_remote_copy(
                    src_ref=amax_ref.at[p], dst_ref=amax_ref.at[p],
                    send_sem=ax_s.at[p], recv_sem=ax_r.at[p],
                    device_id=(0,), device_id_type=pl.DeviceIdType.MESH,
                )
                dummy.wait_send()
                dummy.wait_recv()
        g_amax = jnp.max(amax_ref[:, :, :])
        inv = 127.0 / g_amax
        scale = g_amax / 127.0

        q_cw[:, :] = jnp.clip(
            jnp.round(blk_cw * inv), -127.0, 127.0).astype(jnp.int8)
        out_ref[rows(own_cw), L] = q_cw[:, :].astype(jnp.float32) * scale
        q_cc[:, :] = jnp.clip(
            jnp.round(blk_cc * inv), -127.0, 127.0).astype(jnp.int8)
        out_ref[rows(own_cc), R] = q_cc[:, :].astype(jnp.float32) * scale

        def ag_rdma(h, j):
            slot = h % 2
            ss = pl.ds(j * sub, sub)
            src_cw = q_cw.at[ss] if h == 0 else ag_cw.at[(h - 1) % 2, ss]
            src_cc = q_cc.at[ss] if h == 0 else ag_cc.at[(h - 1) % 2, ss]
            rd_cw = pltpu.make_async_remote_copy(
                src_ref=src_cw, dst_ref=ag_cw.at[slot, ss],
                send_sem=ag_s_cw.at[slot, j], recv_sem=ag_r_cw.at[slot, j],
                device_id=(right,), device_id_type=pl.DeviceIdType.MESH,
            )
            rd_cc = pltpu.make_async_remote_copy(
                src_ref=src_cc, dst_ref=ag_cc.at[slot, ss],
                send_sem=ag_s_cc.at[slot, j], recv_sem=ag_r_cc.at[slot, j],
                device_id=(left,), device_id_type=pl.DeviceIdType.MESH,
            )
            return rd_cw, rd_cc

        for j in range(K_SUB):
            rd_cw, rd_cc = ag_rdma(0, j)
            rd_cw.start()
            rd_cc.start()
        for h in range(N_DEV - 1):
            c_cw = jnp.mod(rp - h, N_DEV)
            c_cc = jnp.mod(rp + h, N_DEV)
            slot = h % 2
            for j in range(K_SUB):
                rd_cw, rd_cc = ag_rdma(h, j)
                rd_cw.wait_recv()
                rd_cc.wait_recv()
                ss = pl.ds(j * sub, sub)
                out_ref[srows(c_cw, j), L] = (
                    ag_cw[slot, ss].astype(jnp.float32) * scale)
                out_ref[srows(c_cc, j), R] = (
                    ag_cc[slot, ss].astype(jnp.float32) * scale)
                if h < N_DEV - 2:
                    if h >= 1:
                        pv_cw, pv_cc = ag_rdma(h - 1, j)
                        pv_cw.wait_send()
                        pv_cc.wait_send()
                    nx_cw, nx_cc = ag_rdma(h + 1, j)
                    nx_cw.start()
                    nx_cc.start()
        for h in (N_DEV - 3, N_DEV - 2):
            for j in range(K_SUB):
                rd_cw, rd_cc = ag_rdma(h, j)
                rd_cw.wait_send()
                rd_cc.wait_send()

    return pl.pallas_call(
        body,
        out_shape=jax.ShapeDtypeStruct((m, n), jnp.float32),
        in_specs=[
            pl.BlockSpec(memory_space=pltpu.SMEM),
            pl.BlockSpec(memory_space=pltpu.VMEM),
            pl.BlockSpec(memory_space=pltpu.VMEM),
        ],
        out_specs=pl.BlockSpec(memory_space=pltpu.VMEM),
        scratch_shapes=[
            pltpu.VMEM((2, ch, n2), jnp.float32),
            pltpu.VMEM((2, ch, n2), jnp.float32),
            pltpu.VMEM((ch, n2), jnp.int8),
            pltpu.VMEM((ch, n2), jnp.int8),
            pltpu.VMEM((2, ch, n2), jnp.int8),
            pltpu.VMEM((2, ch, n2), jnp.int8),
            pltpu.VMEM((N_DEV, 8, 128), jnp.float32),
            pltpu.SemaphoreType.DMA((2, K_SUB)),
            pltpu.SemaphoreType.DMA((2, K_SUB)),
            pltpu.SemaphoreType.DMA((2, K_SUB)),
            pltpu.SemaphoreType.DMA((2, K_SUB)),
            pltpu.SemaphoreType.DMA((2, K_SUB)),
            pltpu.SemaphoreType.DMA((2, K_SUB)),
            pltpu.SemaphoreType.DMA((2, K_SUB)),
            pltpu.SemaphoreType.DMA((2, K_SUB)),
            pltpu.SemaphoreType.DMA((N_DEV,)),
            pltpu.SemaphoreType.DMA((N_DEV,)),
        ],
        compiler_params=pltpu.CompilerParams(
            collective_id=0,
            vmem_limit_bytes=64 * 1024 * 1024,
        ),
    )(params, x, w_mat)


# device time: 267904 ns/iter; 2.9450x vs baseline; 1.0215x over previous
import jax
import jax.numpy as jnp
from jax import lax
from jax.experimental import pallas as pl
from jax.experimental.pallas import tpu as pltpu

N_DEV = 16
K_SUB = 2
K_AG = 8

RING = [0, 1, 5, 9, 13, 14, 10, 6, 2, 3, 7, 11, 15, 12, 8, 4]
POS = [0] * N_DEV
for _p, _id in enumerate(RING):
    POS[_id] = _p


def kernel(x, w_mat):
    m, k = x.shape
    k2, n = w_mat.shape
    assert k == k2
    ch = m // N_DEV
    sub = ch // K_SUB
    sua = ch // K_AG
    n2 = n // 2

    my = lax.axis_index("i")
    ring = jnp.asarray(RING, dtype=jnp.int32)
    rp = jnp.asarray(POS, dtype=jnp.int32)[my]
    right = ring[jnp.mod(rp + 1, N_DEV)]
    left = ring[jnp.mod(rp - 1, N_DEV)]
    params = jnp.stack([rp, left, right]).astype(jnp.int32)

    def body(params_ref, x_ref, w_ref, out_ref,
             comm_cw, comm_cc, q_cw, q_cc, ag_cw, ag_cc, amax_ref,
             rs_s_cw, rs_r_cw, rs_s_cc, rs_r_cc,
             ag_s_cw, ag_r_cw, ag_s_cc, ag_r_cc,
             ax_s, ax_r):
        rp = params_ref[0]
        left = params_ref[1]
        right = params_ref[2]

        def rows(idx):
            return pl.ds(idx * ch, ch)

        def srows(idx, j):
            return pl.ds(idx * ch + j * sub, sub)

        L = pl.ds(0, n2)
        R = pl.ds(n2, n2)

        def gemm_half(c, half):
            out_ref[rows(c), half] = jnp.dot(
                x_ref[rows(c), :], w_ref[:, half],
                preferred_element_type=jnp.float32,
            )

        def rs_rdma(h, j):
            slot = h % 2
            s_cw = jnp.mod(rp - h, N_DEV)
            s_cc = jnp.mod(rp + h, N_DEV)
            rd_cw = pltpu.make_async_remote_copy(
                src_ref=out_ref.at[srows(s_cw, j), L],
                dst_ref=comm_cw.at[slot, pl.ds(j * sub, sub)],
                send_sem=rs_s_cw.at[slot, j], recv_sem=rs_r_cw.at[slot, j],
                device_id=(right,), device_id_type=pl.DeviceIdType.MESH,
            )
            rd_cc = pltpu.make_async_remote_copy(
                src_ref=out_ref.at[srows(s_cc, j), R],
                dst_ref=comm_cc.at[slot, pl.ds(j * sub, sub)],
                send_sem=rs_s_cc.at[slot, j], recv_sem=rs_r_cc.at[slot, j],
                device_id=(left,), device_id_type=pl.DeviceIdType.MESH,
            )
            return rd_cw, rd_cc

        barrier_sem = pltpu.get_barrier_semaphore()
        for nbr in (left, right):
            pl.semaphore_signal(
                barrier_sem, inc=1,
                device_id=(nbr,), device_id_type=pl.DeviceIdType.MESH,
            )
        pl.semaphore_wait(barrier_sem, 2)

        gemm_half(rp, L)
        gemm_half(rp, R)
        for j in range(K_SUB):
            rd_cw, rd_cc = rs_rdma(0, j)
            rd_cw.start()
            rd_cc.start()
        gemm_half(jnp.mod(rp - 1, N_DEV), L)
        gemm_half(jnp.mod(rp + 1, N_DEV), R)

        amax_parts = []
        for h in range(N_DEV - 1):
            r_cw = jnp.mod(rp - h - 1, N_DEV)
            r_cc = jnp.mod(rp + h + 1, N_DEV)
            slot = h % 2
            last = h == N_DEV - 2
            for j in range(K_SUB):
                rd_cw, rd_cc = rs_rdma(h, j)
                rd_cw.wait_recv()
                rd_cc.wait_recv()
                cslice = pl.ds(j * sub, sub)
                acc_cw = out_ref[srows(r_cw, j), L] + comm_cw[slot, cslice]
                acc_cc = out_ref[srows(r_cc, j), R] + comm_cc[slot, cslice]
                if last:
                    acc_cw = jnp.maximum(acc_cw, 0.0)
                    acc_cc = jnp.maximum(acc_cc, 0.0)
                    amax_parts.append(jnp.max(acc_cw))
                    amax_parts.append(jnp.max(acc_cc))
                out_ref[srows(r_cw, j), L] = acc_cw
                out_ref[srows(r_cc, j), R] = acc_cc
                if not last:
                    if h >= 1:
                        pv_cw, pv_cc = rs_rdma(h - 1, j)
                        pv_cw.wait_send()
                        pv_cc.wait_send()
                    nx_cw, nx_cc = rs_rdma(h + 1, j)
                    nx_cw.start()
                    nx_cc.start()
            if h < N_DEV - 2:
                gemm_half(jnp.mod(rp - h - 2, N_DEV), L)
                gemm_half(jnp.mod(rp + h + 2, N_DEV), R)
        for h in (N_DEV - 3, N_DEV - 2):
            for j in range(K_SUB):
                rd_cw, rd_cc = rs_rdma(h, j)
                rd_cw.wait_send()
                rd_cc.wait_send()

        own_cw = jnp.mod(rp + 1, N_DEV)
        own_cc = jnp.mod(rp - 1, N_DEV)
        local_amax = amax_parts[0]
        for part in amax_parts[1:]:
            local_amax = jnp.maximum(local_amax, part)

        amax_ref[rp] = jnp.full((8, 128), local_amax, dtype=jnp.float32)
        for p in range(N_DEV):
            @pl.when(p != rp)
            def _():
                rd = pltpu.make_async_remote_copy(
                    src_ref=amax_ref.at[rp],
                    dst_ref=amax_ref.at[rp],
                    send_sem=ax_s.at[p], recv_sem=ax_r.at[rp],
                    device_id=(RING[p],),
                    device_id_type=pl.DeviceIdType.MESH,
                )
                rd.start()
        for p in range(N_DEV):
            @pl.when(p != rp)
            def _():
                dummy = pltpu.make_async_remote_copy(
                    src_ref=amax_ref.at[p], dst_ref=amax_ref.at[p],
                    send_sem=ax_s.at[p], recv_sem=ax_r.at[p],
                    device_id=(0,), device_id_type=pl.DeviceIdType.MESH,
                )
                dummy.wait_send()
                dummy.wait_recv()
        g_amax = jnp.max(amax_ref[:, :, :])
        inv = 127.0 / g_amax
        scale = g_amax / 127.0

        q_cw[:, :] = jnp.clip(
            jnp.round(out_ref[rows(own_cw), L] * inv),
            -127.0, 127.0).astype(jnp.int8)
        q_cc[:, :] = jnp.clip(
            jnp.round(out_ref[rows(own_cc), R] * inv),
            -127.0, 127.0).astype(jnp.int8)

        def ag_rows(idx, j):
            return pl.ds(idx * ch + j * sua, sua)

        def ag_rdma(h, j):
            slot = h % 2
            ss = pl.ds(j * sua, sua)
            src_cw = q_cw.at[ss] if h == 0 else ag_cw.at[(h - 1) % 2, ss]
            src_cc = q_cc.at[ss] if h == 0 else ag_cc.at[(h - 1) % 2, ss]
            rd_cw = pltpu.make_async_remote_copy(
                src_ref=src_cw, dst_ref=ag_cw.at[slot, ss],
                send_sem=ag_s_cw.at[slot, j], recv_sem=ag_r_cw.at[slot, j],
                device_id=(right,), device_id_type=pl.DeviceIdType.MESH,
            )
            rd_cc = pltpu.make_async_remote_copy(
                src_ref=src_cc, dst_ref=ag_cc.at[slot, ss],
                send_sem=ag_s_cc.at[slot, j], recv_sem=ag_r_cc.at[slot, j],
                device_id=(left,), device_id_type=pl.DeviceIdType.MESH,
            )
            return rd_cw, rd_cc

        for j in range(K_AG):
            rd_cw, rd_cc = ag_rdma(0, j)
            rd_cw.start()
            rd_cc.start()
        out_ref[rows(own_cw), L] = q_cw[:, :].astype(jnp.float32) * scale
        out_ref[rows(own_cc), R] = q_cc[:, :].astype(jnp.float32) * scale

        for h in range(N_DEV - 1):
            c_cw = jnp.mod(rp - h, N_DEV)
            c_cc = jnp.mod(rp + h, N_DEV)
            slot = h % 2
            for j in range(K_AG):
                rd_cw, rd_cc = ag_rdma(h, j)
                rd_cw.wait_recv()
                rd_cc.wait_recv()
                ss = pl.ds(j * sua, sua)
                out_ref[ag_rows(c_cw, j), L] = (
                    ag_cw[slot, ss].astype(jnp.float32) * scale)
                out_ref[ag_rows(c_cc, j), R] = (
                    ag_cc[slot, ss].astype(jnp.float32) * scale)
                if h < N_DEV - 2:
                    if h >= 1:
                        pv_cw, pv_cc = ag_rdma(h - 1, j)
                        pv_cw.wait_send()
                        pv_cc.wait_send()
                    nx_cw, nx_cc = ag_rdma(h + 1, j)
                    nx_cw.start()
                    nx_cc.start()
        for h in (N_DEV - 3, N_DEV - 2):
            for j in range(K_AG):
                rd_cw, rd_cc = ag_rdma(h, j)
                rd_cw.wait_send()
                rd_cc.wait_send()

    return pl.pallas_call(
        body,
        out_shape=jax.ShapeDtypeStruct((m, n), jnp.float32),
        in_specs=[
            pl.BlockSpec(memory_space=pltpu.SMEM),
            pl.BlockSpec(memory_space=pltpu.VMEM),
            pl.BlockSpec(memory_space=pltpu.VMEM),
        ],
        out_specs=pl.BlockSpec(memory_space=pltpu.VMEM),
        scratch_shapes=[
            pltpu.VMEM((2, ch, n2), jnp.float32),
            pltpu.VMEM((2, ch, n2), jnp.float32),
            pltpu.VMEM((ch, n2), jnp.int8),
            pltpu.VMEM((ch, n2), jnp.int8),
            pltpu.VMEM((2, ch, n2), jnp.int8),
            pltpu.VMEM((2, ch, n2), jnp.int8),
            pltpu.VMEM((N_DEV, 8, 128), jnp.float32),
            pltpu.SemaphoreType.DMA((2, K_SUB)),
            pltpu.SemaphoreType.DMA((2, K_SUB)),
            pltpu.SemaphoreType.DMA((2, K_SUB)),
            pltpu.SemaphoreType.DMA((2, K_SUB)),
            pltpu.SemaphoreType.DMA((2, K_AG)),
            pltpu.SemaphoreType.DMA((2, K_AG)),
            pltpu.SemaphoreType.DMA((2, K_AG)),
            pltpu.SemaphoreType.DMA((2, K_AG)),
            pltpu.SemaphoreType.DMA((N_DEV,)),
            pltpu.SemaphoreType.DMA((N_DEV,)),
        ],
        compiler_params=pltpu.CompilerParams(
            collective_id=0,
            vmem_limit_bytes=64 * 1024 * 1024,
        ),
    )(params, x, w_mat)


# device time: 252822 ns/iter; 3.1207x vs baseline; 1.0597x over previous
import jax
import jax.numpy as jnp
from jax import lax
from jax.experimental import pallas as pl
from jax.experimental.pallas import tpu as pltpu

N_DEV = 16
K_SUB = 2
K_AG = 8

RING = [0, 1, 5, 9, 13, 14, 10, 6, 2, 3, 7, 11, 15, 12, 8, 4]


def kernel(x, w_mat):
    m, k = x.shape
    k2, n = w_mat.shape
    assert k == k2
    ch = m // N_DEV
    sub = ch // K_SUB
    sua = ch // K_AG
    n2 = n // 2

    def body(x_ref, w_ref, out_hbm, y_ref,
             comm_cw, comm_cc, q_cw, q_cc, ag_cw, ag_cc, amax_ref,
             rs_s_cw, rs_r_cw, rs_s_cc, rs_r_cc,
             ag_s_cw, ag_r_cw, ag_s_cc, ag_r_cc,
             ax_s, ax_r, st_sem):
        my = lax.axis_index("i")
        rp = jnp.int32(0)
        left = jnp.int32(0)
        right = jnp.int32(0)
        for p in range(N_DEV):
            hit = my == RING[p]
            rp = jnp.where(hit, jnp.int32(p), rp)
            left = jnp.where(hit, jnp.int32(RING[(p - 1) % N_DEV]), left)
            right = jnp.where(hit, jnp.int32(RING[(p + 1) % N_DEV]), right)

        def rows(idx):
            return pl.ds(idx * ch, ch)

        def srows(idx, j):
            return pl.ds(idx * ch + j * sub, sub)

        L = pl.ds(0, n2)
        R = pl.ds(n2, n2)

        def gemm_half(c, half):
            y_ref[rows(c), half] = jnp.dot(
                x_ref[rows(c), :], w_ref[:, half],
                preferred_element_type=jnp.float32,
            )

        def store_out(c, half, hj):
            cp = pltpu.make_async_copy(
                y_ref.at[rows(c), half],
                out_hbm.at[rows(c), half],
                st_sem.at[c, hj],
            )
            cp.start()
            return cp

        def rs_rdma(h, j):
            slot = h % 2
            s_cw = jnp.mod(rp - h, N_DEV)
            s_cc = jnp.mod(rp + h, N_DEV)
            rd_cw = pltpu.make_async_remote_copy(
                src_ref=y_ref.at[srows(s_cw, j), L],
                dst_ref=comm_cw.at[slot, pl.ds(j * sub, sub)],
                send_sem=rs_s_cw.at[slot, j], recv_sem=rs_r_cw.at[slot, j],
                device_id=(right,), device_id_type=pl.DeviceIdType.MESH,
            )
            rd_cc = pltpu.make_async_remote_copy(
                src_ref=y_ref.at[srows(s_cc, j), R],
                dst_ref=comm_cc.at[slot, pl.ds(j * sub, sub)],
                send_sem=rs_s_cc.at[slot, j], recv_sem=rs_r_cc.at[slot, j],
                device_id=(left,), device_id_type=pl.DeviceIdType.MESH,
            )
            return rd_cw, rd_cc

        barrier_sem = pltpu.get_barrier_semaphore()
        for nbr in (left, right):
            pl.semaphore_signal(
                barrier_sem, inc=1,
                device_id=(nbr,), device_id_type=pl.DeviceIdType.MESH,
            )
        pl.semaphore_wait(barrier_sem, 2)

        gemm_half(rp, L)
        gemm_half(rp, R)
        for j in range(K_SUB):
            rd_cw, rd_cc = rs_rdma(0, j)
            rd_cw.start()
            rd_cc.start()
        gemm_half(jnp.mod(rp - 1, N_DEV), L)
        gemm_half(jnp.mod(rp + 1, N_DEV), R)

        amax_parts = []
        for h in range(N_DEV - 1):
            r_cw = jnp.mod(rp - h - 1, N_DEV)
            r_cc = jnp.mod(rp + h + 1, N_DEV)
            slot = h % 2
            last = h == N_DEV - 2
            for j in range(K_SUB):
                rd_cw, rd_cc = rs_rdma(h, j)
                rd_cw.wait_recv()
                rd_cc.wait_recv()
                cslice = pl.ds(j * sub, sub)
                acc_cw = y_ref[srows(r_cw, j), L] + comm_cw[slot, cslice]
                acc_cc = y_ref[srows(r_cc, j), R] + comm_cc[slot, cslice]
                if last:
                    acc_cw = jnp.maximum(acc_cw, 0.0)
                    acc_cc = jnp.maximum(acc_cc, 0.0)
                    amax_parts.append(jnp.max(acc_cw))
                    amax_parts.append(jnp.max(acc_cc))
                y_ref[srows(r_cw, j), L] = acc_cw
                y_ref[srows(r_cc, j), R] = acc_cc
                if not last:
                    if h >= 1:
                        pv_cw, pv_cc = rs_rdma(h - 1, j)
                        pv_cw.wait_send()
                        pv_cc.wait_send()
                    nx_cw, nx_cc = rs_rdma(h + 1, j)
                    nx_cw.start()
                    nx_cc.start()
            if h < N_DEV - 2:
                gemm_half(jnp.mod(rp - h - 2, N_DEV), L)
                gemm_half(jnp.mod(rp + h + 2, N_DEV), R)
        for h in (N_DEV - 3, N_DEV - 2):
            for j in range(K_SUB):
                rd_cw, rd_cc = rs_rdma(h, j)
                rd_cw.wait_send()
                rd_cc.wait_send()

        own_cw = jnp.mod(rp + 1, N_DEV)
        own_cc = jnp.mod(rp - 1, N_DEV)
        local_amax = amax_parts[0]
        for part in amax_parts[1:]:
            local_amax = jnp.maximum(local_amax, part)

        amax_ref[rp] = jnp.full((8, 128), local_amax, dtype=jnp.float32)
        for p in range(N_DEV):
            @pl.when(p != rp)
            def _():
                rd = pltpu.make_async_remote_copy(
                    src_ref=amax_ref.at[rp],
                    dst_ref=amax_ref.at[rp],
                    send_sem=ax_s.at[p], recv_sem=ax_r.at[rp],
                    device_id=(RING[p],),
                    device_id_type=pl.DeviceIdType.MESH,
                )
                rd.start()
        for p in range(N_DEV):
            @pl.when(p != rp)
            def _():
                dummy = pltpu.make_async_remote_copy(
                    src_ref=amax_ref.at[p], dst_ref=amax_ref.at[p],
                    send_sem=ax_s.at[p], recv_sem=ax_r.at[p],
                    device_id=(0,), device_id_type=pl.DeviceIdType.MESH,
                )
                dummy.wait_send()
                dummy.wait_recv()
        g_amax = jnp.max(amax_ref[:, :, :])
        inv = 127.0 / g_amax
        scale = g_amax / 127.0

        q_cw[:, :] = jnp.clip(
            jnp.round(y_ref[rows(own_cw), L] * inv),
            -127.0, 127.0).astype(jnp.int8)
        q_cc[:, :] = jnp.clip(
            jnp.round(y_ref[rows(own_cc), R] * inv),
            -127.0, 127.0).astype(jnp.int8)

        def ag_rows(idx, j):
            return pl.ds(idx * ch + j * sua, sua)

        def ag_rdma(h, j):
            slot = h % 2
            ss = pl.ds(j * sua, sua)
            src_cw = q_cw.at[ss] if h == 0 else ag_cw.at[(h - 1) % 2, ss]
            src_cc = q_cc.at[ss] if h == 0 else ag_cc.at[(h - 1) % 2, ss]
            rd_cw = pltpu.make_async_remote_copy(
                src_ref=src_cw, dst_ref=ag_cw.at[slot, ss],
                send_sem=ag_s_cw.at[slot, j], recv_sem=ag_r_cw.at[slot, j],
                device_id=(right,), device_id_type=pl.DeviceIdType.MESH,
            )
            rd_cc = pltpu.make_async_remote_copy(
                src_ref=src_cc, dst_ref=ag_cc.at[slot, ss],
                send_sem=ag_s_cc.at[slot, j], recv_sem=ag_r_cc.at[slot, j],
                device_id=(left,), device_id_type=pl.DeviceIdType.MESH,
            )
            return rd_cw, rd_cc

        for j in range(K_AG):
            rd_cw, rd_cc = ag_rdma(0, j)
            rd_cw.start()
            rd_cc.start()
        y_ref[rows(own_cw), L] = q_cw[:, :].astype(jnp.float32) * scale
        store_out(own_cw, L, 0)
        y_ref[rows(own_cc), R] = q_cc[:, :].astype(jnp.float32) * scale
        store_out(own_cc, R, 1)

        for h in range(N_DEV - 1):
            c_cw = jnp.mod(rp - h, N_DEV)
            c_cc = jnp.mod(rp + h, N_DEV)
            slot = h % 2
            for j in range(K_AG):
                rd_cw, rd_cc = ag_rdma(h, j)
                rd_cw.wait_recv()
                rd_cc.wait_recv()
                ss = pl.ds(j * sua, sua)
                y_ref[ag_rows(c_cw, j), L] = (
                    ag_cw[slot, ss].astype(jnp.float32) * scale)
                y_ref[ag_rows(c_cc, j), R] = (
                    ag_cc[slot, ss].astype(jnp.float32) * scale)
                if h < N_DEV - 2:
                    if h >= 1:
                        pv_cw, pv_cc = ag_rdma(h - 1, j)
                        pv_cw.wait_send()
                        pv_cc.wait_send()
                    nx_cw, nx_cc = ag_rdma(h + 1, j)
                    nx_cw.start()
                    nx_cc.start()
            store_out(c_cw, L, 0)
            store_out(c_cc, R, 1)
        for h in (N_DEV - 3, N_DEV - 2):
            for j in range(K_AG):
                rd_cw, rd_cc = ag_rdma(h, j)
                rd_cw.wait_send()
                rd_cc.wait_send()
        for c in range(N_DEV):
            for hj, half in ((0, L), (1, R)):
                pltpu.make_async_copy(
                    y_ref.at[rows(c), half],
                    out_hbm.at[rows(c), half],
                    st_sem.at[c, hj],
                ).wait()

    return pl.pallas_call(
        body,
        out_shape=jax.ShapeDtypeStruct((m, n), jnp.float32),
        in_specs=[
            pl.BlockSpec(memory_space=pltpu.VMEM),
            pl.BlockSpec(memory_space=pltpu.VMEM),
        ],
        out_specs=pl.BlockSpec(memory_space=pltpu.MemorySpace.HBM),
        scratch_shapes=[
            pltpu.VMEM((m, n), jnp.float32),
            pltpu.VMEM((2, ch, n2), jnp.float32),
            pltpu.VMEM((2, ch, n2), jnp.float32),
            pltpu.VMEM((ch, n2), jnp.int8),
            pltpu.VMEM((ch, n2), jnp.int8),
            pltpu.VMEM((2, ch, n2), jnp.int8),
            pltpu.VMEM((2, ch, n2), jnp.int8),
            pltpu.VMEM((N_DEV, 8, 128), jnp.float32),
            pltpu.SemaphoreType.DMA((2, K_SUB)),
            pltpu.SemaphoreType.DMA((2, K_SUB)),
            pltpu.SemaphoreType.DMA((2, K_SUB)),
            pltpu.SemaphoreType.DMA((2, K_SUB)),
            pltpu.SemaphoreType.DMA((2, K_AG)),
            pltpu.SemaphoreType.DMA((2, K_AG)),
            pltpu.SemaphoreType.DMA((2, K_AG)),
            pltpu.SemaphoreType.DMA((2, K_AG)),
            pltpu.SemaphoreType.DMA((N_DEV,)),
            pltpu.SemaphoreType.DMA((N_DEV,)),
            pltpu.SemaphoreType.DMA((N_DEV, 2)),
        ],
        compiler_params=pltpu.CompilerParams(
            collective_id=0,
            vmem_limit_bytes=64 * 1024 * 1024,
        ),
    )(x, w_mat)
